# baseline (speedup 1.0000x reference)
import functools

import jax
import jax.numpy as jnp
from jax import lax
from jax.experimental import pallas as pl
from jax.experimental.pallas import tpu as pltpu
from jax.experimental.pallas import tpu_sc as plsc

N = 10002
E = 320064
HID = 128
NOUT = N // 6

NC, NS, LN = 2, 16, 16
NW = NC * NS
BLK = 128
NBLK = 80
EW = NBLK * BLK
EP = NW * EW
NR = 10240
RPT = NR // NS
VPAD = 12288
NO_PAD = 2048

f32 = jnp.float32
i32 = jnp.int32


def _rsqrt16(x):
    i = plsc.bitcast(x, i32)
    i = jnp.int32(0x5F3759DF) - lax.shift_right_logical(i, 1)
    y = plsc.bitcast(i, f32)
    for _ in range(3):
        y = y * (1.5 - 0.5 * x * y * y)
    return y


def _mesh():
    return plsc.VectorSubcoreMesh(
        core_axis_name="c", subcore_axis_name="s", num_cores=NC, num_subcores=NS
    )


_SC_PARAMS = dict(compiler_params=pltpu.CompilerParams(needs_layout_passes=False))


def _make_prep():
    @functools.partial(
        pl.kernel,
        out_type=[
            jax.ShapeDtypeStruct((NR, HID), f32),
            jax.ShapeDtypeStruct((NR,), f32),
        ],
        mesh=_mesh(),
        scratch_types=[
            pltpu.VMEM((2, NBLK, BLK), i32),
            pltpu.VMEM((2, NBLK, BLK), f32),
            pltpu.VMEM((1, BLK), f32),
            pltpu.VMEM((NR,), f32),
            pltpu.VMEM((NR,), f32),
            pltpu.VMEM((RPT,), f32),
            pltpu.VMEM((LN, HID), f32),
            pltpu.VMEM_SHARED((NR,), f32),
            pltpu.VMEM_SHARED((NR,), f32),
        ],
        **_SC_PARAMS,
    )
    def prep(dst3, w3, d_o, dinv5_o,
             dst_v, w_v, one_v, degv, cntv, zv, dbuf,
             deg_sh, cnt_sh):
        cid = lax.axis_index("c")
        sid = lax.axis_index("s")
        ones16 = jnp.full((LN,), 1.0, f32)
        zeros16 = jnp.zeros((LN,), f32)
        for k in range(BLK // LN):
            one_v[0, pl.ds(k * LN, LN)] = ones16

        def zv_body(k, carry):
            zv[pl.ds(k * LN, LN)] = zeros16
            return carry

        lax.fori_loop(0, RPT // LN, zv_body, 0)
        pltpu.sync_copy(zv, deg_sh.at[pl.ds(sid * RPT, RPT)])
        pltpu.sync_copy(zv, cnt_sh.at[pl.ds(sid * RPT, RPT)])
        pltpu.sync_copy(dst3.at[pl.ds(sid * 2, 2)], dst_v)
        pltpu.sync_copy(w3.at[pl.ds(sid * 2, 2)], w_v)
        plsc.subcore_barrier()
        for a in range(2):
            def deg_body(j, carry, a=a):
                pltpu.sync_copy(w_v.at[a, j], deg_sh.at[dst_v.at[a, j]], add=True)
                pltpu.sync_copy(one_v.at[0], cnt_sh.at[dst_v.at[a, j]], add=True)
                return carry

            lax.fori_loop(0, NBLK, deg_body, 0)
        plsc.subcore_barrier()
        pltpu.sync_copy(deg_sh, degv)
        pltpu.sync_copy(cnt_sh, cntv)

        def dinv_body(k, carry):
            sl = pl.ds(k * LN, LN)
            degv[sl] = _rsqrt16(degv[sl] + 1.0)
            cntv[sl] = _rsqrt16(cntv[sl] + 1.0)
            return carry

        lax.fori_loop(0, NR // LN, dinv_body, 0)

        @pl.when(cid == 0)
        def _():
            pltpu.sync_copy(cntv.at[pl.ds(sid * RPT, RPT)],
                            dinv5_o.at[pl.ds(sid * RPT, RPT)])

            def drow(m, carry):
                base = sid * RPT + m * LN
                d16 = degv[pl.ds(base, LN)]
                for rr in range(LN):
                    s = d16[rr]
                    for k in range(HID // LN):
                        dbuf[rr, pl.ds(k * LN, LN)] = s * ones16
                pltpu.sync_copy(dbuf, d_o.at[pl.ds(base, LN)])
                return carry

            lax.fori_loop(0, RPT // LN, drow, 0)

    return prep


def _make_mp():
    @functools.partial(
        pl.kernel,
        out_type=jax.ShapeDtypeStruct((NC, NR, HID), f32),
        mesh=_mesh(),
        scratch_types=[
            pltpu.VMEM((NBLK, BLK), i32),
            pltpu.VMEM((NBLK, BLK), i32),
            pltpu.VMEM((NBLK, BLK), f32),
            pltpu.VMEM((BLK, HID), f32),
            pltpu.VMEM((1, NBLK), i32),
            pltpu.VMEM((1, BLK), i32),
            pltpu.VMEM_SHARED((NR, HID), f32),
            pltpu.SemaphoreType.DMA,
            pltpu.SemaphoreType.DMA,
        ],
        **_SC_PARAMS,
    )
    def mp(g, src2, dst2, w2, s_out,
           src_v, dst_v, w_v, rb0, idxw, idxb, acc, sem0, sem1):
        cid = lax.axis_index("c")
        sid = lax.axis_index("s")
        wid = cid * NS + sid
        iot = lax.iota(i32, LN)
        for k in range(NBLK // LN):
            idxw[0, pl.ds(k * LN, LN)] = wid * NBLK + k * LN + iot
        pltpu.async_copy(src2.at[idxw.at[0]], src_v, sem0).wait()
        pltpu.async_copy(dst2.at[idxw.at[0]], dst_v, sem0).wait()
        pltpu.async_copy(w2.at[idxw.at[0]], w_v, sem0).wait()
        kc = jnp.where(cid == 0, jnp.float32(1.0), jnp.float32(0.0))

        def initblk(jb, carry):
            base = sid * RPT + jb * BLK
            for k in range(BLK // LN):
                idxb[0, pl.ds(k * LN, LN)] = base + k * LN + iot
            pltpu.async_copy(g.at[idxb.at[0]], rb0, sem0).wait()

            def srow(m, carry2):
                for k in range(HID // LN):
                    sl = pl.ds(k * LN, LN)
                    rb0[m, sl] = rb0[m, sl] * kc
                return carry2

            lax.fori_loop(0, BLK, srow, 0)
            pltpu.sync_copy(rb0, acc.at[idxb.at[0]])
            return carry

        lax.fori_loop(0, RPT // BLK, initblk, 0)
        plsc.subcore_barrier()

        def scale(rb, j):
            def grp(m, carry):
                c16 = w_v[j, pl.ds(m * LN, LN)]
                for rr in range(LN):
                    sc = c16[rr]
                    r = m * LN + rr
                    for k in range(HID // LN):
                        sl = pl.ds(k * LN, LN)
                        rb[r, sl] = rb[r, sl] * sc
                return carry

            lax.fori_loop(0, BLK // LN, grp, 0)

        def step(j, carry):
            pltpu.async_copy(g.at[src_v.at[j]], rb0, sem0).wait()
            scale(rb0, j)
            pltpu.sync_copy(rb0, acc.at[dst_v.at[j]], add=True)
            return carry

        lax.fori_loop(0, NBLK, step, 0)
        plsc.subcore_barrier()
        sl2 = pl.ds(sid * RPT, RPT)
        pltpu.sync_copy(acc.at[sl2], s_out.at[cid, sl2])

    return mp


def _make_sc5():
    @functools.partial(
        pl.kernel,
        out_type=[
            jax.ShapeDtypeStruct((NO_PAD,), f32),
            jax.ShapeDtypeStruct((NR,), f32),
        ],
        mesh=_mesh(),
        scratch_types=[
            pltpu.VMEM((VPAD,), f32),
            pltpu.VMEM((NR,), f32),
            pltpu.VMEM((NR,), f32),
            pltpu.VMEM((NBLK, BLK), i32),
            pltpu.VMEM((NBLK, BLK), i32),
            pltpu.VMEM((1, BLK), f32),
            pltpu.VMEM((LN,), f32),
            pltpu.VMEM((LN,), f32),
            pltpu.VMEM((LN,), f32),
            pltpu.VMEM((1, BLK), f32),
            pltpu.VMEM_SHARED((NR,), f32),
        ],
        **_SC_PARAMS,
    )
    def sc5(h5, dinv5, src3, dst3, b5v, lwv, lbv, outp, p5b,
            g5v, d5v, accv, src_v, dst_v, val_v, b5_v, lw_v, lb_v, ov, acc5):
        cid = lax.axis_index("c")
        sid = lax.axis_index("s")
        pltpu.sync_copy(h5, g5v.at[pl.ds(0, NR)])
        pltpu.sync_copy(dinv5, d5v)
        pltpu.sync_copy(b5v, b5_v)
        pltpu.sync_copy(lwv, lw_v)
        pltpu.sync_copy(lbv, lb_v)
        zeros16 = jnp.zeros((LN,), f32)

        def gz(k, carry):
            sl = pl.ds(k * LN, LN)
            g5v[sl] = g5v[sl] * d5v[sl]
            return carry

        lax.fori_loop(0, NR // LN, gz, 0)

        def gz2(k, carry):
            g5v[pl.ds(NR + k * LN, LN)] = zeros16
            return carry

        lax.fori_loop(0, (VPAD - NR) // LN, gz2, 0)
        sl0 = pl.ds(sid * RPT, RPT)
        pltpu.sync_copy(g5v.at[sl0], acc5.at[sl0])
        plsc.subcore_barrier()
        for a in range(2):
            wrow = sid * 2 + a
            pltpu.sync_copy(src3.at[wrow], src_v)
            pltpu.sync_copy(dst3.at[wrow], dst_v)
            ebase = wrow * EW

            def eblk(j, carry, ebase=ebase):
                for k in range(BLK // LN):
                    sl = pl.ds(k * LN, LN)
                    sidx = src_v[j, sl]
                    g = plsc.load_gather(g5v, [sidx])
                    gidx = ebase + j * BLK + k * LN + lax.iota(i32, LN)
                    val_v[0, sl] = jnp.where(gidx < E, g, 0.0)
                pltpu.sync_copy(val_v.at[0], acc5.at[dst_v.at[j]], add=True)
                return carry

            lax.fori_loop(0, NBLK, eblk, 0)
        plsc.subcore_barrier()

        @pl.when(cid == 0)
        def _():
            pltpu.sync_copy(acc5.at[sl0], p5b.at[sl0])
            plsc.subcore_barrier()
            pltpu.sync_copy(p5b, accv)

            def vb(k, carry):
                sl = pl.ds(k * LN, LN)
                g5v[sl] = d5v[sl] * accv[sl] + b5_v[...]
                return carry

            lax.fori_loop(0, NR // LN, vb, 0)

            lw16 = lw_v[...]
            for t in range(BLK // LN):
                i16 = sid * BLK + t * LN + lax.iota(i32, LN)
                base6 = i16 * 6
                a16 = lb_v[...]
                for j in range(6):
                    g = plsc.load_gather(g5v, [base6 + j])
                    a16 = a16 + g * lw16[j]
                ov[0, pl.ds(t * LN, LN)] = a16
            pltpu.sync_copy(ov.at[0], outp.at[pl.ds(sid * BLK, BLK)])

    return sc5


def _tc1_body(x_ref, w_ref, d_ref, o_ref):
    o_ref[...] = d_ref[...] * jnp.dot(x_ref[...], w_ref[...],
                                      preferred_element_type=f32)


def _tc1(x_pad, w1p, D):
    RB = 1024
    return pl.pallas_call(
        _tc1_body,
        grid=(NR // RB,),
        in_specs=[pl.BlockSpec((RB, 8), lambda i: (i, 0)),
                  pl.BlockSpec((8, HID), lambda i: (0, 0)),
                  pl.BlockSpec((RB, HID), lambda i: (i, 0))],
        out_specs=pl.BlockSpec((RB, HID), lambda i: (i, 0)),
        out_shape=jax.ShapeDtypeStruct((NR, HID), f32),
    )(x_pad, w1p, D)


def _tcl_body(s_ref, d_ref, b_ref, a_ref, w_ref, o_ref):
    xsum = d_ref[...] * (s_ref[0] + s_ref[1]) + b_ref[...]
    a = a_ref[0]
    x = jnp.where(xsum >= 0, xsum, a * xsum)
    o_ref[...] = d_ref[...] * jnp.dot(x, w_ref[...],
                                      preferred_element_type=f32)


def _tc_layer(S, D, b, a_arr, W):
    RB = 1024
    return pl.pallas_call(
        _tcl_body,
        grid=(NR // RB,),
        in_specs=[pl.BlockSpec((NC, RB, HID), lambda i: (0, i, 0)),
                  pl.BlockSpec((RB, HID), lambda i: (i, 0)),
                  pl.BlockSpec((1, HID), lambda i: (0, 0)),
                  pl.BlockSpec(memory_space=pltpu.SMEM),
                  pl.BlockSpec((HID, HID), lambda i: (0, 0))],
        out_specs=pl.BlockSpec((RB, HID), lambda i: (i, 0)),
        out_shape=jax.ShapeDtypeStruct((NR, HID), f32),
    )(S, D, b.reshape(1, HID), a_arr, W)


def _tc4_body(s_ref, d_ref, b_ref, a_ref, w_ref, o_ref):
    xsum = d_ref[...] * (s_ref[0] + s_ref[1]) + b_ref[...]
    a = a_ref[0]
    x = jnp.where(xsum >= 0, xsum, a * xsum)
    o_ref[0] = lax.dot_general(w_ref[...], x, (((1,), (1,)), ((), ())),
                               preferred_element_type=f32)


def _tc4(S, D, b, a_arr, w5t):
    return pl.pallas_call(
        _tc4_body,
        grid=(NR // BLK,),
        in_specs=[pl.BlockSpec((NC, BLK, HID), lambda i: (0, i, 0)),
                  pl.BlockSpec((BLK, HID), lambda i: (i, 0)),
                  pl.BlockSpec((1, HID), lambda i: (0, 0)),
                  pl.BlockSpec(memory_space=pltpu.SMEM),
                  pl.BlockSpec((1, HID), lambda i: (0, 0))],
        out_specs=pl.BlockSpec((1, 1, HID), lambda i: (i, 0, 0)),
        out_shape=jax.ShapeDtypeStruct((NR // BLK, 1, HID), f32),
    )(S, D, b.reshape(1, HID), a_arr, w5t)


def kernel(x, edge_index, edge_weight, W1, b1, W2, b2, W3, b3, W4, b4,
           W5, b5, prelu_a, lin_W, lin_b):
    pad = EP - E
    padrow = (N + (jnp.arange(pad, dtype=i32) % (NR - N))).astype(i32)
    src3 = jnp.concatenate([edge_index[0], padrow]).reshape(NW, NBLK, BLK)
    dst3 = jnp.concatenate([edge_index[1], padrow]).reshape(NW, NBLK, BLK)
    w3 = jnp.concatenate(
        [edge_weight, jnp.zeros((pad,), f32)]).reshape(NW, NBLK, BLK)
    x_pad = jnp.zeros((NR, 8), f32).at[:N, :3].set(x)
    w1p = jnp.zeros((8, HID), f32).at[:3].set(W1)
    a_arr = prelu_a.reshape(1)
    b5v = jnp.full((LN,), b5[0], f32)
    lwv = jnp.zeros((LN,), f32).at[:6].set(lin_W[:, 0])
    lbv = jnp.full((LN,), lin_b[0], f32)

    src2 = src3.reshape(NW * NBLK, BLK)
    dst2 = dst3.reshape(NW * NBLK, BLK)
    w2 = w3.reshape(NW * NBLK, BLK)

    prep = _make_prep()
    mp = _make_mp()
    sc5 = _make_sc5()

    D, dinv5 = prep(dst3, w3)
    G = _tc1(x_pad, w1p, D)
    S = mp(G, src2, dst2, w2)
    G = _tc_layer(S, D, b1, a_arr, W2)
    S = mp(G, src2, dst2, w2)
    G = _tc_layer(S, D, b2, a_arr, W3)
    S = mp(G, src2, dst2, w2)
    G = _tc_layer(S, D, b3, a_arr, W4)
    S = mp(G, src2, dst2, w2)
    h5 = _tc4(S, D, b4, a_arr, W5.reshape(1, HID)).reshape(NR)
    outp, _ = sc5(h5, dinv5, src3, dst3, b5v, lwv, lbv)
    return outp[:NOUT]


# speedup vs baseline: 1.0006x; 1.0006x over previous
import functools

import jax
import jax.numpy as jnp
from jax import lax
from jax.experimental import pallas as pl
from jax.experimental.pallas import tpu as pltpu
from jax.experimental.pallas import tpu_sc as plsc

N = 10002
E = 320064
HID = 128
NOUT = N // 6

NC, NS, LN = 2, 16, 16
NW = NC * NS
BLK = 128
NBLK = 80
EW = NBLK * BLK
EP = NW * EW
NR = 10240
RPT = NR // NS
VPAD = 12288
NO_PAD = 2048

f32 = jnp.float32
i32 = jnp.int32


def _rsqrt16(x):
    i = plsc.bitcast(x, i32)
    i = jnp.int32(0x5F3759DF) - lax.shift_right_logical(i, 1)
    y = plsc.bitcast(i, f32)
    for _ in range(3):
        y = y * (1.5 - 0.5 * x * y * y)
    return y


def _mesh():
    return plsc.VectorSubcoreMesh(
        core_axis_name="c", subcore_axis_name="s", num_cores=NC, num_subcores=NS
    )


_SC_PARAMS = dict(compiler_params=pltpu.CompilerParams(needs_layout_passes=False))


def _make_prep():
    @functools.partial(
        pl.kernel,
        out_type=[
            jax.ShapeDtypeStruct((NR, HID), f32),
            jax.ShapeDtypeStruct((NR,), f32),
        ],
        mesh=_mesh(),
        scratch_types=[
            pltpu.VMEM((2, NBLK, BLK), i32),
            pltpu.VMEM((2, NBLK, BLK), f32),
            pltpu.VMEM((1, BLK), f32),
            pltpu.VMEM((NR,), f32),
            pltpu.VMEM((NR,), f32),
            pltpu.VMEM((RPT,), f32),
            pltpu.VMEM((LN, HID), f32),
            pltpu.VMEM_SHARED((NR,), f32),
            pltpu.VMEM_SHARED((NR,), f32),
        ],
        **_SC_PARAMS,
    )
    def prep(dst3, w3, d_o, dinv5_o,
             dst_v, w_v, one_v, degv, cntv, zv, dbuf,
             deg_sh, cnt_sh):
        cid = lax.axis_index("c")
        sid = lax.axis_index("s")
        ones16 = jnp.full((LN,), 1.0, f32)
        zeros16 = jnp.zeros((LN,), f32)
        for k in range(BLK // LN):
            one_v[0, pl.ds(k * LN, LN)] = ones16

        def zv_body(k, carry):
            zv[pl.ds(k * LN, LN)] = zeros16
            return carry

        lax.fori_loop(0, RPT // LN, zv_body, 0)
        pltpu.sync_copy(zv, deg_sh.at[pl.ds(sid * RPT, RPT)])
        pltpu.sync_copy(zv, cnt_sh.at[pl.ds(sid * RPT, RPT)])
        pltpu.sync_copy(dst3.at[pl.ds(sid * 2, 2)], dst_v)
        pltpu.sync_copy(w3.at[pl.ds(sid * 2, 2)], w_v)
        plsc.subcore_barrier()
        for a in range(2):
            def deg_body(j, carry, a=a):
                pltpu.sync_copy(w_v.at[a, j], deg_sh.at[dst_v.at[a, j]], add=True)
                pltpu.sync_copy(one_v.at[0], cnt_sh.at[dst_v.at[a, j]], add=True)
                return carry

            lax.fori_loop(0, NBLK, deg_body, 0)
        plsc.subcore_barrier()
        pltpu.sync_copy(deg_sh, degv)
        pltpu.sync_copy(cnt_sh, cntv)

        def dinv_body(k, carry):
            sl = pl.ds(k * LN, LN)
            degv[sl] = _rsqrt16(degv[sl] + 1.0)
            cntv[sl] = _rsqrt16(cntv[sl] + 1.0)
            return carry

        lax.fori_loop(0, NR // LN, dinv_body, 0)

        @pl.when(cid == 0)
        def _():
            pltpu.sync_copy(cntv.at[pl.ds(sid * RPT, RPT)],
                            dinv5_o.at[pl.ds(sid * RPT, RPT)])

            def drow(m, carry):
                base = sid * RPT + m * LN
                d16 = degv[pl.ds(base, LN)]
                for rr in range(LN):
                    s = d16[rr]
                    for k in range(HID // LN):
                        dbuf[rr, pl.ds(k * LN, LN)] = s * ones16
                pltpu.sync_copy(dbuf, d_o.at[pl.ds(base, LN)])
                return carry

            lax.fori_loop(0, RPT // LN, drow, 0)

    return prep


def _make_mp():
    @functools.partial(
        pl.kernel,
        out_type=jax.ShapeDtypeStruct((NC, NR, HID), f32),
        mesh=_mesh(),
        scratch_types=[
            pltpu.VMEM((NBLK, BLK), i32),
            pltpu.VMEM((NBLK, BLK), i32),
            pltpu.VMEM((NBLK, BLK), f32),
            pltpu.VMEM((BLK, HID), f32),
            pltpu.VMEM((1, NBLK), i32),
            pltpu.VMEM((1, BLK), i32),
            pltpu.VMEM_SHARED((NR, HID), f32),
            pltpu.SemaphoreType.DMA,
        ],
        **_SC_PARAMS,
    )
    def mp(g, src2, dst2, w2, s_out,
           src_v, dst_v, w_v, rb0, idxw, idxb, acc, sem0):
        cid = lax.axis_index("c")
        sid = lax.axis_index("s")
        wid = cid * NS + sid
        iot = lax.iota(i32, LN)
        for k in range(NBLK // LN):
            idxw[0, pl.ds(k * LN, LN)] = wid * NBLK + k * LN + iot
        pltpu.async_copy(src2.at[idxw.at[0]], src_v, sem0).wait()
        pltpu.async_copy(dst2.at[idxw.at[0]], dst_v, sem0).wait()
        pltpu.async_copy(w2.at[idxw.at[0]], w_v, sem0).wait()
        kc = jnp.where(cid == 0, jnp.float32(1.0), jnp.float32(0.0))

        def initblk(jb, carry):
            base = sid * RPT + jb * BLK
            for k in range(BLK // LN):
                idxb[0, pl.ds(k * LN, LN)] = base + k * LN + iot
            pltpu.async_copy(g.at[idxb.at[0]], rb0, sem0).wait()

            def srow(m, carry2):
                for k in range(HID // LN):
                    sl = pl.ds(k * LN, LN)
                    rb0[m, sl] = rb0[m, sl] * kc
                return carry2

            lax.fori_loop(0, BLK, srow, 0)
            pltpu.sync_copy(rb0, acc.at[idxb.at[0]])
            return carry

        lax.fori_loop(0, RPT // BLK, initblk, 0)
        plsc.subcore_barrier()

        def scale(rb, j):
            def grp(m, carry):
                c16 = w_v[j, pl.ds(m * LN, LN)]
                for rr in range(LN):
                    sc = c16[rr]
                    r = m * LN + rr
                    for k in range(HID // LN):
                        sl = pl.ds(k * LN, LN)
                        rb[r, sl] = rb[r, sl] * sc
                return carry

            lax.fori_loop(0, BLK // LN, grp, 0)

        def step(j, carry):
            pltpu.async_copy(g.at[src_v.at[j]], rb0, sem0).wait()
            scale(rb0, j)
            pltpu.sync_copy(rb0, acc.at[dst_v.at[j]], add=True)
            return carry

        lax.fori_loop(0, NBLK, step, 0)
        plsc.subcore_barrier()
        sl2 = pl.ds(sid * RPT, RPT)
        pltpu.sync_copy(acc.at[sl2], s_out.at[cid, sl2])

    return mp


def _make_sc5():
    @functools.partial(
        pl.kernel,
        out_type=[
            jax.ShapeDtypeStruct((NO_PAD,), f32),
            jax.ShapeDtypeStruct((NR,), f32),
        ],
        mesh=_mesh(),
        scratch_types=[
            pltpu.VMEM((VPAD,), f32),
            pltpu.VMEM((NR,), f32),
            pltpu.VMEM((NR,), f32),
            pltpu.VMEM((NBLK, BLK), i32),
            pltpu.VMEM((NBLK, BLK), i32),
            pltpu.VMEM((1, BLK), f32),
            pltpu.VMEM((LN,), f32),
            pltpu.VMEM((LN,), f32),
            pltpu.VMEM((LN,), f32),
            pltpu.VMEM((1, BLK), f32),
            pltpu.VMEM_SHARED((NR,), f32),
        ],
        **_SC_PARAMS,
    )
    def sc5(h5, dinv5, src3, dst3, b5v, lwv, lbv, outp, p5b,
            g5v, d5v, accv, src_v, dst_v, val_v, b5_v, lw_v, lb_v, ov, acc5):
        cid = lax.axis_index("c")
        sid = lax.axis_index("s")
        pltpu.sync_copy(h5, g5v.at[pl.ds(0, NR)])
        pltpu.sync_copy(dinv5, d5v)
        pltpu.sync_copy(b5v, b5_v)
        pltpu.sync_copy(lwv, lw_v)
        pltpu.sync_copy(lbv, lb_v)
        zeros16 = jnp.zeros((LN,), f32)

        def gz(k, carry):
            sl = pl.ds(k * LN, LN)
            g5v[sl] = g5v[sl] * d5v[sl]
            return carry

        lax.fori_loop(0, NR // LN, gz, 0)

        def gz2(k, carry):
            g5v[pl.ds(NR + k * LN, LN)] = zeros16
            return carry

        lax.fori_loop(0, (VPAD - NR) // LN, gz2, 0)
        sl0 = pl.ds(sid * RPT, RPT)
        pltpu.sync_copy(g5v.at[sl0], acc5.at[sl0])
        plsc.subcore_barrier()
        for a in range(2):
            wrow = sid * 2 + a
            pltpu.sync_copy(src3.at[wrow], src_v)
            pltpu.sync_copy(dst3.at[wrow], dst_v)
            ebase = wrow * EW

            def eblk(j, carry, ebase=ebase):
                for k in range(BLK // LN):
                    sl = pl.ds(k * LN, LN)
                    sidx = src_v[j, sl]
                    g = plsc.load_gather(g5v, [sidx])
                    gidx = ebase + j * BLK + k * LN + lax.iota(i32, LN)
                    val_v[0, sl] = jnp.where(gidx < E, g, 0.0)
                pltpu.sync_copy(val_v.at[0], acc5.at[dst_v.at[j]], add=True)
                return carry

            lax.fori_loop(0, NBLK, eblk, 0)
        plsc.subcore_barrier()

        @pl.when(cid == 0)
        def _():
            pltpu.sync_copy(acc5.at[sl0], p5b.at[sl0])
            plsc.subcore_barrier()
            pltpu.sync_copy(p5b, accv)

            def vb(k, carry):
                sl = pl.ds(k * LN, LN)
                g5v[sl] = d5v[sl] * accv[sl] + b5_v[...]
                return carry

            lax.fori_loop(0, NR // LN, vb, 0)

            lw16 = lw_v[...]
            for t in range(BLK // LN):
                i16 = sid * BLK + t * LN + lax.iota(i32, LN)
                base6 = i16 * 6
                a16 = lb_v[...]
                for j in range(6):
                    g = plsc.load_gather(g5v, [base6 + j])
                    a16 = a16 + g * lw16[j]
                ov[0, pl.ds(t * LN, LN)] = a16
            pltpu.sync_copy(ov.at[0], outp.at[pl.ds(sid * BLK, BLK)])

    return sc5


def _tc1_body(x_ref, w_ref, d_ref, o_ref):
    o_ref[...] = d_ref[...] * jnp.dot(x_ref[...], w_ref[...],
                                      preferred_element_type=f32)


def _tc1(x_pad, w1p, D):
    RB = 1024
    return pl.pallas_call(
        _tc1_body,
        grid=(NR // RB,),
        in_specs=[pl.BlockSpec((RB, 8), lambda i: (i, 0)),
                  pl.BlockSpec((8, HID), lambda i: (0, 0)),
                  pl.BlockSpec((RB, HID), lambda i: (i, 0))],
        out_specs=pl.BlockSpec((RB, HID), lambda i: (i, 0)),
        out_shape=jax.ShapeDtypeStruct((NR, HID), f32),
    )(x_pad, w1p, D)


def _tcl_body(s_ref, d_ref, b_ref, a_ref, w_ref, o_ref):
    xsum = d_ref[...] * (s_ref[0] + s_ref[1]) + b_ref[...]
    a = a_ref[0]
    x = jnp.where(xsum >= 0, xsum, a * xsum)
    o_ref[...] = d_ref[...] * jnp.dot(x, w_ref[...],
                                      preferred_element_type=f32)


def _tc_layer(S, D, b, a_arr, W):
    RB = 1024
    return pl.pallas_call(
        _tcl_body,
        grid=(NR // RB,),
        in_specs=[pl.BlockSpec((NC, RB, HID), lambda i: (0, i, 0)),
                  pl.BlockSpec((RB, HID), lambda i: (i, 0)),
                  pl.BlockSpec((1, HID), lambda i: (0, 0)),
                  pl.BlockSpec(memory_space=pltpu.SMEM),
                  pl.BlockSpec((HID, HID), lambda i: (0, 0))],
        out_specs=pl.BlockSpec((RB, HID), lambda i: (i, 0)),
        out_shape=jax.ShapeDtypeStruct((NR, HID), f32),
    )(S, D, b.reshape(1, HID), a_arr, W)


def _tc4_body(s_ref, d_ref, b_ref, a_ref, w_ref, o_ref):
    xsum = d_ref[...] * (s_ref[0] + s_ref[1]) + b_ref[...]
    a = a_ref[0]
    x = jnp.where(xsum >= 0, xsum, a * xsum)
    o_ref[0] = lax.dot_general(w_ref[...], x, (((1,), (1,)), ((), ())),
                               preferred_element_type=f32)


def _tc4(S, D, b, a_arr, w5t):
    return pl.pallas_call(
        _tc4_body,
        grid=(NR // BLK,),
        in_specs=[pl.BlockSpec((NC, BLK, HID), lambda i: (0, i, 0)),
                  pl.BlockSpec((BLK, HID), lambda i: (i, 0)),
                  pl.BlockSpec((1, HID), lambda i: (0, 0)),
                  pl.BlockSpec(memory_space=pltpu.SMEM),
                  pl.BlockSpec((1, HID), lambda i: (0, 0))],
        out_specs=pl.BlockSpec((1, 1, HID), lambda i: (i, 0, 0)),
        out_shape=jax.ShapeDtypeStruct((NR // BLK, 1, HID), f32),
    )(S, D, b.reshape(1, HID), a_arr, w5t)


def kernel(x, edge_index, edge_weight, W1, b1, W2, b2, W3, b3, W4, b4,
           W5, b5, prelu_a, lin_W, lin_b):
    pad = EP - E
    padrow = (N + (jnp.arange(pad, dtype=i32) % (NR - N))).astype(i32)
    src3 = jnp.concatenate([edge_index[0], padrow]).reshape(NW, NBLK, BLK)
    dst3 = jnp.concatenate([edge_index[1], padrow]).reshape(NW, NBLK, BLK)
    w3 = jnp.concatenate(
        [edge_weight, jnp.zeros((pad,), f32)]).reshape(NW, NBLK, BLK)
    x_pad = jnp.zeros((NR, 8), f32).at[:N, :3].set(x)
    w1p = jnp.zeros((8, HID), f32).at[:3].set(W1)
    a_arr = prelu_a.reshape(1)
    b5v = jnp.full((LN,), b5[0], f32)
    lwv = jnp.zeros((LN,), f32).at[:6].set(lin_W[:, 0])
    lbv = jnp.full((LN,), lin_b[0], f32)

    src2 = src3.reshape(NW * NBLK, BLK)
    dst2 = dst3.reshape(NW * NBLK, BLK)
    w2 = w3.reshape(NW * NBLK, BLK)

    prep = _make_prep()
    mp = _make_mp()
    sc5 = _make_sc5()

    D, dinv5 = prep(dst3, w3)
    G = _tc1(x_pad, w1p, D)
    S = mp(G, src2, dst2, w2)
    G = _tc_layer(S, D, b1, a_arr, W2)
    S = mp(G, src2, dst2, w2)
    G = _tc_layer(S, D, b2, a_arr, W3)
    S = mp(G, src2, dst2, w2)
    G = _tc_layer(S, D, b3, a_arr, W4)
    S = mp(G, src2, dst2, w2)
    h5 = _tc4(S, D, b4, a_arr, W5.reshape(1, HID)).reshape(NR)
    outp, _ = sc5(h5, dinv5, src3, dst3, b5v, lwv, lbv)
    return outp[:NOUT]


# speedup vs baseline: 1.0319x; 1.0312x over previous
import functools

import jax
import jax.numpy as jnp
from jax import lax
from jax.experimental import pallas as pl
from jax.experimental.pallas import tpu as pltpu
from jax.experimental.pallas import tpu_sc as plsc

N = 10002
E = 320064
HID = 128
NOUT = N // 6

NC, NS, LN = 2, 16, 16
NW = NC * NS
BLK = 128
NBLK = 80
EW = NBLK * BLK
EP = NW * EW
NR = 10240
RPT = NR // NS
VPAD = 12288
NO_PAD = 2048

f32 = jnp.float32
i32 = jnp.int32


def _rsqrt16(x):
    i = plsc.bitcast(x, i32)
    i = jnp.int32(0x5F3759DF) - lax.shift_right_logical(i, 1)
    y = plsc.bitcast(i, f32)
    for _ in range(3):
        y = y * (1.5 - 0.5 * x * y * y)
    return y


def _mesh():
    return plsc.VectorSubcoreMesh(
        core_axis_name="c", subcore_axis_name="s", num_cores=NC, num_subcores=NS
    )


_SC_PARAMS = dict(compiler_params=pltpu.CompilerParams(needs_layout_passes=False))


def _make_prep():
    @functools.partial(
        pl.kernel,
        out_type=[
            jax.ShapeDtypeStruct((NR, HID), f32),
            jax.ShapeDtypeStruct((NR,), f32),
        ],
        mesh=_mesh(),
        scratch_types=[
            pltpu.VMEM((2, NBLK, BLK), i32),
            pltpu.VMEM((2, NBLK, BLK), f32),
            pltpu.VMEM((1, BLK), f32),
            pltpu.VMEM((NR,), f32),
            pltpu.VMEM((NR,), f32),
            pltpu.VMEM((RPT,), f32),
            pltpu.VMEM((LN, HID), f32),
            pltpu.VMEM_SHARED((NR,), f32),
            pltpu.VMEM_SHARED((NR,), f32),
        ],
        **_SC_PARAMS,
    )
    def prep(dst3, w3, d_o, dinv5_o,
             dst_v, w_v, one_v, degv, cntv, zv, dbuf,
             deg_sh, cnt_sh):
        cid = lax.axis_index("c")
        sid = lax.axis_index("s")
        ones16 = jnp.full((LN,), 1.0, f32)
        zeros16 = jnp.zeros((LN,), f32)
        for k in range(BLK // LN):
            one_v[0, pl.ds(k * LN, LN)] = ones16

        def zv_body(k, carry):
            zv[pl.ds(k * LN, LN)] = zeros16
            return carry

        lax.fori_loop(0, RPT // LN, zv_body, 0)
        pltpu.sync_copy(zv, deg_sh.at[pl.ds(sid * RPT, RPT)])
        pltpu.sync_copy(zv, cnt_sh.at[pl.ds(sid * RPT, RPT)])
        pltpu.sync_copy(dst3.at[pl.ds(sid * 2, 2)], dst_v)
        pltpu.sync_copy(w3.at[pl.ds(sid * 2, 2)], w_v)
        plsc.subcore_barrier()
        for a in range(2):
            def deg_body(j, carry, a=a):
                pltpu.sync_copy(w_v.at[a, j], deg_sh.at[dst_v.at[a, j]], add=True)
                pltpu.sync_copy(one_v.at[0], cnt_sh.at[dst_v.at[a, j]], add=True)
                return carry

            lax.fori_loop(0, NBLK, deg_body, 0)
        plsc.subcore_barrier()
        pltpu.sync_copy(deg_sh, degv)
        pltpu.sync_copy(cnt_sh, cntv)

        def dinv_body(k, carry):
            sl = pl.ds(k * LN, LN)
            degv[sl] = _rsqrt16(degv[sl] + 1.0)
            cntv[sl] = _rsqrt16(cntv[sl] + 1.0)
            return carry

        lax.fori_loop(0, NR // LN, dinv_body, 0)

        @pl.when(cid == 0)
        def _():
            pltpu.sync_copy(cntv.at[pl.ds(sid * RPT, RPT)],
                            dinv5_o.at[pl.ds(sid * RPT, RPT)])

            def drow(m, carry):
                base = sid * RPT + m * LN
                d16 = degv[pl.ds(base, LN)]
                for rr in range(LN):
                    s = d16[rr]
                    for k in range(HID // LN):
                        dbuf[rr, pl.ds(k * LN, LN)] = s * ones16
                pltpu.sync_copy(dbuf, d_o.at[pl.ds(base, LN)])
                return carry

            lax.fori_loop(0, RPT // LN, drow, 0)

    return prep


def _make_mp():
    @functools.partial(
        pl.kernel,
        out_type=jax.ShapeDtypeStruct((NC, NR, HID), f32),
        mesh=_mesh(),
        scratch_types=[
            pltpu.VMEM((NBLK, BLK), i32),
            pltpu.VMEM((NBLK, BLK), i32),
            pltpu.VMEM((NBLK, BLK), f32),
            pltpu.VMEM((BLK, HID), f32),
            pltpu.VMEM((1, NBLK), i32),
            pltpu.VMEM((1, BLK), i32),
            pltpu.VMEM_SHARED((NR, HID), f32),
            pltpu.SemaphoreType.DMA,
        ],
        **_SC_PARAMS,
    )
    def mp(g, src2, dst2, w2, s_out,
           src_v, dst_v, w_v, rb0, idxw, idxb, acc, sem0):
        cid = lax.axis_index("c")
        sid = lax.axis_index("s")
        wid = cid * NS + sid
        iot = lax.iota(i32, LN)
        for k in range(NBLK // LN):
            idxw[0, pl.ds(k * LN, LN)] = wid * NBLK + k * LN + iot
        pltpu.async_copy(src2.at[idxw.at[0]], src_v, sem0).wait()
        pltpu.async_copy(dst2.at[idxw.at[0]], dst_v, sem0).wait()
        pltpu.async_copy(w2.at[idxw.at[0]], w_v, sem0).wait()
        kc = jnp.where(cid == 0, jnp.float32(1.0), jnp.float32(0.0))

        def initblk(jb, carry):
            base = sid * RPT + jb * BLK
            for k in range(BLK // LN):
                idxb[0, pl.ds(k * LN, LN)] = base + k * LN + iot
            pltpu.async_copy(g.at[idxb.at[0]], rb0, sem0).wait()

            def srow(m, carry2):
                for k in range(HID // LN):
                    sl = pl.ds(k * LN, LN)
                    rb0[m, sl] = rb0[m, sl] * kc
                return carry2

            lax.fori_loop(0, BLK, srow, 0)
            pltpu.sync_copy(rb0, acc.at[idxb.at[0]])
            return carry

        lax.fori_loop(0, RPT // BLK, initblk, 0)
        plsc.subcore_barrier()

        def scale(rb, j):
            def grp(m, carry):
                c16 = w_v[j, pl.ds(m * LN, LN)]
                for rr in range(LN):
                    sc = c16[rr]
                    r = m * LN + rr
                    for k in range(HID // LN):
                        sl = pl.ds(k * LN, LN)
                        rb[r, sl] = rb[r, sl] * sc
                return carry

            lax.fori_loop(0, BLK // LN, grp, 0)

        def step(j, carry):
            pltpu.async_copy(g.at[src_v.at[j]], rb0, sem0).wait()
            scale(rb0, j)
            pltpu.sync_copy(rb0, acc.at[dst_v.at[j]], add=True)
            return carry

        lax.fori_loop(0, NBLK, step, 0)
        plsc.subcore_barrier()
        sl2 = pl.ds(sid * RPT, RPT)
        pltpu.sync_copy(acc.at[sl2], s_out.at[cid, sl2])

    return mp


def _make_sc5():
    @functools.partial(
        pl.kernel,
        out_type=[
            jax.ShapeDtypeStruct((NO_PAD,), f32),
            jax.ShapeDtypeStruct((NR,), f32),
        ],
        mesh=_mesh(),
        scratch_types=[
            pltpu.VMEM((VPAD,), f32),
            pltpu.VMEM((NR,), f32),
            pltpu.VMEM((NR,), f32),
            pltpu.VMEM((NBLK, BLK), i32),
            pltpu.VMEM((NBLK, BLK), i32),
            pltpu.VMEM((1, BLK), f32),
            pltpu.VMEM((LN,), f32),
            pltpu.VMEM((LN,), f32),
            pltpu.VMEM((LN,), f32),
            pltpu.VMEM((1, BLK), f32),
            pltpu.VMEM_SHARED((NR,), f32),
        ],
        **_SC_PARAMS,
    )
    def sc5(h5, dinv5, src3, dst3, b5v, lwv, lbv, outp, p5b,
            g5v, d5v, accv, src_v, dst_v, val_v, b5_v, lw_v, lb_v, ov, acc5):
        cid = lax.axis_index("c")
        sid = lax.axis_index("s")
        pltpu.sync_copy(h5, g5v.at[pl.ds(0, NR)])
        pltpu.sync_copy(dinv5, d5v)
        pltpu.sync_copy(b5v, b5_v)
        pltpu.sync_copy(lwv, lw_v)
        pltpu.sync_copy(lbv, lb_v)
        zeros16 = jnp.zeros((LN,), f32)

        def gz(k, carry):
            sl = pl.ds(k * LN, LN)
            g5v[sl] = g5v[sl] * d5v[sl]
            return carry

        lax.fori_loop(0, NR // LN, gz, 0)

        def gz2(k, carry):
            g5v[pl.ds(NR + k * LN, LN)] = zeros16
            return carry

        lax.fori_loop(0, (VPAD - NR) // LN, gz2, 0)
        sl0 = pl.ds(sid * RPT, RPT)
        pltpu.sync_copy(g5v.at[sl0], acc5.at[sl0])
        plsc.subcore_barrier()
        for a in range(2):
            wrow = sid * 2 + a
            pltpu.sync_copy(src3.at[wrow], src_v)
            pltpu.sync_copy(dst3.at[wrow], dst_v)
            ebase = wrow * EW

            def eblk(j, carry, ebase=ebase):
                for k in range(BLK // LN):
                    sl = pl.ds(k * LN, LN)
                    sidx = src_v[j, sl]
                    g = plsc.load_gather(g5v, [sidx])
                    gidx = ebase + j * BLK + k * LN + lax.iota(i32, LN)
                    val_v[0, sl] = jnp.where(gidx < E, g, 0.0)
                pltpu.sync_copy(val_v.at[0], acc5.at[dst_v.at[j]], add=True)
                return carry

            lax.fori_loop(0, NBLK, eblk, 0)
        plsc.subcore_barrier()

        @pl.when(cid == 0)
        def _():
            pltpu.sync_copy(acc5.at[sl0], p5b.at[sl0])
            plsc.subcore_barrier()
            pltpu.sync_copy(p5b, accv)

            def vb(k, carry):
                sl = pl.ds(k * LN, LN)
                g5v[sl] = d5v[sl] * accv[sl] + b5_v[...]
                return carry

            lax.fori_loop(0, NR // LN, vb, 0)

            lw16 = lw_v[...]
            for t in range(BLK // LN):
                i16 = sid * BLK + t * LN + lax.iota(i32, LN)
                base6 = i16 * 6
                a16 = lb_v[...]
                for j in range(6):
                    g = plsc.load_gather(g5v, [base6 + j])
                    a16 = a16 + g * lw16[j]
                ov[0, pl.ds(t * LN, LN)] = a16
            pltpu.sync_copy(ov.at[0], outp.at[pl.ds(sid * BLK, BLK)])

    return sc5


def _tc1_body(x_ref, w_ref, d_ref, o_ref):
    o_ref[...] = d_ref[...] * jnp.dot(x_ref[...], w_ref[...],
                                      preferred_element_type=f32)


def _tc1(x_pad, w1p, D):
    RB = 1024
    return pl.pallas_call(
        _tc1_body,
        grid=(NR // RB,),
        in_specs=[pl.BlockSpec((RB, 8), lambda i: (i, 0)),
                  pl.BlockSpec((8, HID), lambda i: (0, 0)),
                  pl.BlockSpec((RB, HID), lambda i: (i, 0))],
        out_specs=pl.BlockSpec((RB, HID), lambda i: (i, 0)),
        out_shape=jax.ShapeDtypeStruct((NR, HID), f32),
    )(x_pad, w1p, D)


def _tcl_body(s_ref, d_ref, b_ref, a_ref, w_ref, o_ref):
    xsum = d_ref[...] * (s_ref[0] + s_ref[1]) + b_ref[...]
    a = a_ref[0]
    x = jnp.where(xsum >= 0, xsum, a * xsum)
    o_ref[...] = d_ref[...] * jnp.dot(x, w_ref[...],
                                      preferred_element_type=f32)


def _tc_layer(S, D, b, a_arr, W):
    RB = 1024
    return pl.pallas_call(
        _tcl_body,
        grid=(NR // RB,),
        in_specs=[pl.BlockSpec((NC, RB, HID), lambda i: (0, i, 0)),
                  pl.BlockSpec((RB, HID), lambda i: (i, 0)),
                  pl.BlockSpec((1, HID), lambda i: (0, 0)),
                  pl.BlockSpec(memory_space=pltpu.SMEM),
                  pl.BlockSpec((HID, HID), lambda i: (0, 0))],
        out_specs=pl.BlockSpec((RB, HID), lambda i: (i, 0)),
        out_shape=jax.ShapeDtypeStruct((NR, HID), f32),
    )(S, D, b.reshape(1, HID), a_arr, W)


def _tc4_body(s_ref, d_ref, b_ref, a_ref, w_ref, o_ref):
    xsum = d_ref[...] * (s_ref[0] + s_ref[1]) + b_ref[...]
    a = a_ref[0]
    x = jnp.where(xsum >= 0, xsum, a * xsum)
    for t in range(8):
        o_ref[t] = lax.dot_general(w_ref[...], x[t * BLK:(t + 1) * BLK, :],
                                   (((1,), (1,)), ((), ())),
                                   preferred_element_type=f32)


def _tc4(S, D, b, a_arr, w5t):
    RB = 1024
    return pl.pallas_call(
        _tc4_body,
        grid=(NR // RB,),
        in_specs=[pl.BlockSpec((NC, RB, HID), lambda i: (0, i, 0)),
                  pl.BlockSpec((RB, HID), lambda i: (i, 0)),
                  pl.BlockSpec((1, HID), lambda i: (0, 0)),
                  pl.BlockSpec(memory_space=pltpu.SMEM),
                  pl.BlockSpec((1, HID), lambda i: (0, 0))],
        out_specs=pl.BlockSpec((8, 1, HID), lambda i: (i, 0, 0)),
        out_shape=jax.ShapeDtypeStruct((NR // BLK, 1, HID), f32),
    )(S, D, b.reshape(1, HID), a_arr, w5t)


def kernel(x, edge_index, edge_weight, W1, b1, W2, b2, W3, b3, W4, b4,
           W5, b5, prelu_a, lin_W, lin_b):
    pad = EP - E
    padrow = (N + (jnp.arange(pad, dtype=i32) % (NR - N))).astype(i32)
    src3 = jnp.concatenate([edge_index[0], padrow]).reshape(NW, NBLK, BLK)
    dst3 = jnp.concatenate([edge_index[1], padrow]).reshape(NW, NBLK, BLK)
    w3 = jnp.concatenate(
        [edge_weight, jnp.zeros((pad,), f32)]).reshape(NW, NBLK, BLK)
    x_pad = jnp.zeros((NR, 8), f32).at[:N, :3].set(x)
    w1p = jnp.zeros((8, HID), f32).at[:3].set(W1)
    a_arr = prelu_a.reshape(1)
    b5v = jnp.full((LN,), b5[0], f32)
    lwv = jnp.zeros((LN,), f32).at[:6].set(lin_W[:, 0])
    lbv = jnp.full((LN,), lin_b[0], f32)

    src2 = src3.reshape(NW * NBLK, BLK)
    dst2 = dst3.reshape(NW * NBLK, BLK)
    w2 = w3.reshape(NW * NBLK, BLK)

    prep = _make_prep()
    mp = _make_mp()
    sc5 = _make_sc5()

    D, dinv5 = prep(dst3, w3)
    G = _tc1(x_pad, w1p, D)
    S = mp(G, src2, dst2, w2)
    G = _tc_layer(S, D, b1, a_arr, W2)
    S = mp(G, src2, dst2, w2)
    G = _tc_layer(S, D, b2, a_arr, W3)
    S = mp(G, src2, dst2, w2)
    G = _tc_layer(S, D, b3, a_arr, W4)
    S = mp(G, src2, dst2, w2)
    h5 = _tc4(S, D, b4, a_arr, W5.reshape(1, HID)).reshape(NR)
    outp, _ = sc5(h5, dinv5, src3, dst3, b5v, lwv, lbv)
    return outp[:NOUT]


# speedup vs baseline: 1.0369x; 1.0049x over previous
import functools

import jax
import jax.numpy as jnp
from jax import lax
from jax.experimental import pallas as pl
from jax.experimental.pallas import tpu as pltpu
from jax.experimental.pallas import tpu_sc as plsc

N = 10002
E = 320064
HID = 128
NOUT = N // 6

NC, NS, LN = 2, 16, 16
NW = NC * NS
BLK = 128
NBLK = 80
EW = NBLK * BLK
EP = NW * EW
NR = 10240
RPT = NR // NS
VPAD = 12288
NO_PAD = 2048

f32 = jnp.float32
i32 = jnp.int32


def _rsqrt16(x):
    i = plsc.bitcast(x, i32)
    i = jnp.int32(0x5F3759DF) - lax.shift_right_logical(i, 1)
    y = plsc.bitcast(i, f32)
    for _ in range(3):
        y = y * (1.5 - 0.5 * x * y * y)
    return y


def _mesh():
    return plsc.VectorSubcoreMesh(
        core_axis_name="c", subcore_axis_name="s", num_cores=NC, num_subcores=NS
    )


_SC_PARAMS = dict(compiler_params=pltpu.CompilerParams(needs_layout_passes=False))


def _make_prep():
    @functools.partial(
        pl.kernel,
        out_type=[
            jax.ShapeDtypeStruct((NR, HID), f32),
            jax.ShapeDtypeStruct((NR,), f32),
        ],
        mesh=_mesh(),
        scratch_types=[
            pltpu.VMEM((2, NBLK, BLK), i32),
            pltpu.VMEM((2, NBLK, BLK), f32),
            pltpu.VMEM((1, BLK), f32),
            pltpu.VMEM((NR,), f32),
            pltpu.VMEM((NR,), f32),
            pltpu.VMEM((RPT,), f32),
            pltpu.VMEM((LN, HID), f32),
            pltpu.VMEM_SHARED((NR,), f32),
            pltpu.VMEM_SHARED((NR,), f32),
        ],
        **_SC_PARAMS,
    )
    def prep(dst3, w3, d_o, dinv5_o,
             dst_v, w_v, one_v, degv, cntv, zv, dbuf,
             deg_sh, cnt_sh):
        cid = lax.axis_index("c")
        sid = lax.axis_index("s")
        ones16 = jnp.full((LN,), 1.0, f32)
        zeros16 = jnp.zeros((LN,), f32)
        for k in range(BLK // LN):
            one_v[0, pl.ds(k * LN, LN)] = ones16

        def zv_body(k, carry):
            zv[pl.ds(k * LN, LN)] = zeros16
            return carry

        lax.fori_loop(0, RPT // LN, zv_body, 0)
        pltpu.sync_copy(zv, deg_sh.at[pl.ds(sid * RPT, RPT)])
        pltpu.sync_copy(zv, cnt_sh.at[pl.ds(sid * RPT, RPT)])
        pltpu.sync_copy(dst3.at[pl.ds(sid * 2, 2)], dst_v)
        pltpu.sync_copy(w3.at[pl.ds(sid * 2, 2)], w_v)
        plsc.subcore_barrier()
        for a in range(2):
            def deg_body(j, carry, a=a):
                pltpu.sync_copy(w_v.at[a, j], deg_sh.at[dst_v.at[a, j]], add=True)
                pltpu.sync_copy(one_v.at[0], cnt_sh.at[dst_v.at[a, j]], add=True)
                return carry

            lax.fori_loop(0, NBLK, deg_body, 0)
        plsc.subcore_barrier()
        pltpu.sync_copy(deg_sh, degv)
        pltpu.sync_copy(cnt_sh, cntv)

        def dinv_body(k, carry):
            sl = pl.ds(k * LN, LN)
            degv[sl] = _rsqrt16(degv[sl] + 1.0)
            cntv[sl] = _rsqrt16(cntv[sl] + 1.0)
            return carry

        lax.fori_loop(0, NR // LN, dinv_body, 0)

        @pl.when(cid == 0)
        def _():
            pltpu.sync_copy(cntv.at[pl.ds(sid * RPT, RPT)],
                            dinv5_o.at[pl.ds(sid * RPT, RPT)])

            def drow(m, carry):
                base = sid * RPT + m * LN
                d16 = degv[pl.ds(base, LN)]
                for rr in range(LN):
                    s = d16[rr]
                    for k in range(HID // LN):
                        dbuf[rr, pl.ds(k * LN, LN)] = s * ones16
                pltpu.sync_copy(dbuf, d_o.at[pl.ds(base, LN)])
                return carry

            lax.fori_loop(0, RPT // LN, drow, 0)

    return prep


def _make_mp():
    @functools.partial(
        pl.kernel,
        out_type=jax.ShapeDtypeStruct((NC, NR, HID), f32),
        mesh=_mesh(),
        scratch_types=[
            pltpu.VMEM((NBLK, BLK), i32),
            pltpu.VMEM((NBLK, BLK), i32),
            pltpu.VMEM((NBLK, BLK), f32),
            pltpu.VMEM((BLK, HID), f32),
            pltpu.VMEM((1, NBLK), i32),
            pltpu.VMEM((1, BLK), i32),
            pltpu.VMEM_SHARED((NR, HID), f32),
            pltpu.SemaphoreType.DMA,
        ],
        **_SC_PARAMS,
    )
    def mp(g, src2, dst2, w2, s_out,
           src_v, dst_v, w_v, rb0, idxw, idxb, acc, sem0):
        cid = lax.axis_index("c")
        sid = lax.axis_index("s")
        wid = cid * NS + sid
        iot = lax.iota(i32, LN)
        for k in range(NBLK // LN):
            idxw[0, pl.ds(k * LN, LN)] = wid * NBLK + k * LN + iot
        pltpu.async_copy(src2.at[idxw.at[0]], src_v, sem0).wait()
        pltpu.async_copy(dst2.at[idxw.at[0]], dst_v, sem0).wait()
        pltpu.async_copy(w2.at[idxw.at[0]], w_v, sem0).wait()
        kc = jnp.where(cid == 0, jnp.float32(1.0), jnp.float32(0.0))

        def initblk(jb, carry):
            base = sid * RPT + jb * BLK
            for k in range(BLK // LN):
                idxb[0, pl.ds(k * LN, LN)] = base + k * LN + iot
            pltpu.async_copy(g.at[idxb.at[0]], rb0, sem0).wait()

            def srow(m, carry2):
                for k in range(HID // LN):
                    sl = pl.ds(k * LN, LN)
                    rb0[m, sl] = rb0[m, sl] * kc
                return carry2

            lax.fori_loop(0, BLK, srow, 0)
            pltpu.sync_copy(rb0, acc.at[idxb.at[0]])
            return carry

        lax.fori_loop(0, RPT // BLK, initblk, 0)
        plsc.subcore_barrier()

        def scale(rb, j):
            def grp(m, carry):
                c16 = w_v[j, pl.ds(m * LN, LN)]
                for rr in range(LN):
                    sc = c16[rr]
                    r = m * LN + rr
                    for k in range(HID // LN):
                        sl = pl.ds(k * LN, LN)
                        rb[r, sl] = rb[r, sl] * sc
                return carry

            lax.fori_loop(0, BLK // LN, grp, 0)

        def step(j, carry):
            pltpu.async_copy(g.at[src_v.at[j]], rb0, sem0).wait()
            scale(rb0, j)
            pltpu.sync_copy(rb0, acc.at[dst_v.at[j]], add=True)
            return carry

        lax.fori_loop(0, NBLK, step, 0)
        plsc.subcore_barrier()
        sl2 = pl.ds(sid * RPT, RPT)
        pltpu.sync_copy(acc.at[sl2], s_out.at[cid, sl2])

    return mp


def _make_sc5():
    @functools.partial(
        pl.kernel,
        out_type=[
            jax.ShapeDtypeStruct((NO_PAD,), f32),
            jax.ShapeDtypeStruct((NR,), f32),
        ],
        mesh=_mesh(),
        scratch_types=[
            pltpu.VMEM((VPAD,), f32),
            pltpu.VMEM((NR,), f32),
            pltpu.VMEM((NR,), f32),
            pltpu.VMEM((NBLK, BLK), i32),
            pltpu.VMEM((NBLK, BLK), i32),
            pltpu.VMEM((1, BLK), f32),
            pltpu.VMEM((LN,), f32),
            pltpu.VMEM((LN,), f32),
            pltpu.VMEM((LN,), f32),
            pltpu.VMEM((1, BLK), f32),
            pltpu.VMEM_SHARED((NR,), f32),
        ],
        **_SC_PARAMS,
    )
    def sc5(h5, dinv5, src3, dst3, b5v, lwv, lbv, outp, p5b,
            g5v, d5v, accv, src_v, dst_v, val_v, b5_v, lw_v, lb_v, ov, acc5):
        cid = lax.axis_index("c")
        sid = lax.axis_index("s")
        pltpu.sync_copy(h5, g5v.at[pl.ds(0, NR)])
        pltpu.sync_copy(dinv5, d5v)
        pltpu.sync_copy(b5v, b5_v)
        pltpu.sync_copy(lwv, lw_v)
        pltpu.sync_copy(lbv, lb_v)
        zeros16 = jnp.zeros((LN,), f32)

        def gz(k, carry):
            sl = pl.ds(k * LN, LN)
            g5v[sl] = g5v[sl] * d5v[sl]
            return carry

        lax.fori_loop(0, NR // LN, gz, 0)

        def gz2(k, carry):
            g5v[pl.ds(NR + k * LN, LN)] = zeros16
            return carry

        lax.fori_loop(0, (VPAD - NR) // LN, gz2, 0)
        sl0 = pl.ds(sid * RPT, RPT)
        pltpu.sync_copy(g5v.at[sl0], acc5.at[sl0])
        plsc.subcore_barrier()
        for a in range(2):
            wrow = sid * 2 + a
            pltpu.sync_copy(src3.at[wrow], src_v)
            pltpu.sync_copy(dst3.at[wrow], dst_v)
            ebase = wrow * EW

            def eblk(j, carry, ebase=ebase):
                for k in range(BLK // LN):
                    sl = pl.ds(k * LN, LN)
                    sidx = src_v[j, sl]
                    g = plsc.load_gather(g5v, [sidx])
                    gidx = ebase + j * BLK + k * LN + lax.iota(i32, LN)
                    val_v[0, sl] = jnp.where(gidx < E, g, 0.0)
                pltpu.sync_copy(val_v.at[0], acc5.at[dst_v.at[j]], add=True)
                return carry

            lax.fori_loop(0, NBLK, eblk, 0)
        plsc.subcore_barrier()

        @pl.when(cid == 0)
        def _():
            pltpu.sync_copy(acc5.at[sl0], p5b.at[sl0])
            plsc.subcore_barrier()
            pltpu.sync_copy(p5b, accv)

            def vb(k, carry):
                sl = pl.ds(k * LN, LN)
                g5v[sl] = d5v[sl] * accv[sl] + b5_v[...]
                return carry

            lax.fori_loop(0, NR // LN, vb, 0)

            lw16 = lw_v[...]
            for t in range(BLK // LN):
                i16 = sid * BLK + t * LN + lax.iota(i32, LN)
                base6 = i16 * 6
                a16 = lb_v[...]
                for j in range(6):
                    g = plsc.load_gather(g5v, [base6 + j])
                    a16 = a16 + g * lw16[j]
                ov[0, pl.ds(t * LN, LN)] = a16
            pltpu.sync_copy(ov.at[0], outp.at[pl.ds(sid * BLK, BLK)])

    return sc5


def _tc1_body(x_ref, w_ref, d_ref, o_ref):
    o_ref[...] = d_ref[...] * jnp.dot(x_ref[...], w_ref[...],
                                      preferred_element_type=f32)


def _tc1(x_pad, w1p, D):
    RB = 1024
    return pl.pallas_call(
        _tc1_body,
        grid=(NR // RB,),
        in_specs=[pl.BlockSpec((RB, 8), lambda i: (i, 0)),
                  pl.BlockSpec((8, HID), lambda i: (0, 0)),
                  pl.BlockSpec((RB, HID), lambda i: (i, 0))],
        out_specs=pl.BlockSpec((RB, HID), lambda i: (i, 0)),
        out_shape=jax.ShapeDtypeStruct((NR, HID), f32),
    )(x_pad, w1p, D)


def _tcl_body(s_ref, d_ref, b_ref, a_ref, w_ref, o_ref):
    xsum = d_ref[...] * (s_ref[0] + s_ref[1]) + b_ref[...]
    a = a_ref[0]
    x = jnp.where(xsum >= 0, xsum, a * xsum)
    o_ref[...] = d_ref[...] * jnp.dot(x, w_ref[...],
                                      preferred_element_type=f32)


def _tc_layer(S, D, b, a_arr, W):
    RB = 2048
    return pl.pallas_call(
        _tcl_body,
        grid=(NR // RB,),
        in_specs=[pl.BlockSpec((NC, RB, HID), lambda i: (0, i, 0)),
                  pl.BlockSpec((RB, HID), lambda i: (i, 0)),
                  pl.BlockSpec((1, HID), lambda i: (0, 0)),
                  pl.BlockSpec(memory_space=pltpu.SMEM),
                  pl.BlockSpec((HID, HID), lambda i: (0, 0))],
        out_specs=pl.BlockSpec((RB, HID), lambda i: (i, 0)),
        out_shape=jax.ShapeDtypeStruct((NR, HID), f32),
    )(S, D, b.reshape(1, HID), a_arr, W)


def _tc4_body(s_ref, d_ref, b_ref, a_ref, w_ref, o_ref):
    xsum = d_ref[...] * (s_ref[0] + s_ref[1]) + b_ref[...]
    a = a_ref[0]
    x = jnp.where(xsum >= 0, xsum, a * xsum)
    for t in range(8):
        o_ref[t] = lax.dot_general(w_ref[...], x[t * BLK:(t + 1) * BLK, :],
                                   (((1,), (1,)), ((), ())),
                                   preferred_element_type=f32)


def _tc4(S, D, b, a_arr, w5t):
    RB = 1024
    return pl.pallas_call(
        _tc4_body,
        grid=(NR // RB,),
        in_specs=[pl.BlockSpec((NC, RB, HID), lambda i: (0, i, 0)),
                  pl.BlockSpec((RB, HID), lambda i: (i, 0)),
                  pl.BlockSpec((1, HID), lambda i: (0, 0)),
                  pl.BlockSpec(memory_space=pltpu.SMEM),
                  pl.BlockSpec((1, HID), lambda i: (0, 0))],
        out_specs=pl.BlockSpec((8, 1, HID), lambda i: (i, 0, 0)),
        out_shape=jax.ShapeDtypeStruct((NR // BLK, 1, HID), f32),
    )(S, D, b.reshape(1, HID), a_arr, w5t)


def kernel(x, edge_index, edge_weight, W1, b1, W2, b2, W3, b3, W4, b4,
           W5, b5, prelu_a, lin_W, lin_b):
    pad = EP - E
    padrow = (N + (jnp.arange(pad, dtype=i32) % (NR - N))).astype(i32)
    src3 = jnp.concatenate([edge_index[0], padrow]).reshape(NW, NBLK, BLK)
    dst3 = jnp.concatenate([edge_index[1], padrow]).reshape(NW, NBLK, BLK)
    w3 = jnp.concatenate(
        [edge_weight, jnp.zeros((pad,), f32)]).reshape(NW, NBLK, BLK)
    x_pad = jnp.zeros((NR, 8), f32).at[:N, :3].set(x)
    w1p = jnp.zeros((8, HID), f32).at[:3].set(W1)
    a_arr = prelu_a.reshape(1)
    b5v = jnp.full((LN,), b5[0], f32)
    lwv = jnp.zeros((LN,), f32).at[:6].set(lin_W[:, 0])
    lbv = jnp.full((LN,), lin_b[0], f32)

    src2 = src3.reshape(NW * NBLK, BLK)
    dst2 = dst3.reshape(NW * NBLK, BLK)
    w2 = w3.reshape(NW * NBLK, BLK)

    prep = _make_prep()
    mp = _make_mp()
    sc5 = _make_sc5()

    D, dinv5 = prep(dst3, w3)
    G = _tc1(x_pad, w1p, D)
    S = mp(G, src2, dst2, w2)
    G = _tc_layer(S, D, b1, a_arr, W2)
    S = mp(G, src2, dst2, w2)
    G = _tc_layer(S, D, b2, a_arr, W3)
    S = mp(G, src2, dst2, w2)
    G = _tc_layer(S, D, b3, a_arr, W4)
    S = mp(G, src2, dst2, w2)
    h5 = _tc4(S, D, b4, a_arr, W5.reshape(1, HID)).reshape(NR)
    outp, _ = sc5(h5, dinv5, src3, dst3, b5v, lwv, lbv)
    return outp[:NOUT]


# speedup vs baseline: 1.0581x; 1.0204x over previous
import functools

import jax
import jax.numpy as jnp
from jax import lax
from jax.experimental import pallas as pl
from jax.experimental.pallas import tpu as pltpu
from jax.experimental.pallas import tpu_sc as plsc

N = 10002
E = 320064
HID = 128
NOUT = N // 6

NC, NS, LN = 2, 16, 16
NW = NC * NS
BLK = 128
NBLK = 80
EW = NBLK * BLK
EP = NW * EW
NR = 10240
RPT = NR // NS
VPAD = 12288
NO_PAD = 2048

f32 = jnp.float32
i32 = jnp.int32


def _rsqrt16(x):
    i = plsc.bitcast(x, i32)
    i = jnp.int32(0x5F3759DF) - lax.shift_right_logical(i, 1)
    y = plsc.bitcast(i, f32)
    for _ in range(3):
        y = y * (1.5 - 0.5 * x * y * y)
    return y


def _mesh():
    return plsc.VectorSubcoreMesh(
        core_axis_name="c", subcore_axis_name="s", num_cores=NC, num_subcores=NS
    )


_SC_PARAMS = dict(compiler_params=pltpu.CompilerParams(needs_layout_passes=False))


def _make_prep():
    @functools.partial(
        pl.kernel,
        out_type=[
            jax.ShapeDtypeStruct((NR, HID), f32),
            jax.ShapeDtypeStruct((NR,), f32),
        ],
        mesh=_mesh(),
        scratch_types=[
            pltpu.VMEM((2, NBLK, BLK), i32),
            pltpu.VMEM((2, NBLK, BLK), f32),
            pltpu.VMEM((1, BLK), f32),
            pltpu.VMEM((NR,), f32),
            pltpu.VMEM((NR,), f32),
            pltpu.VMEM((RPT,), f32),
            pltpu.VMEM((LN, HID), f32),
            pltpu.VMEM_SHARED((NR,), f32),
            pltpu.VMEM_SHARED((NR,), f32),
        ],
        **_SC_PARAMS,
    )
    def prep(dst3, w3, d_o, dinv5_o,
             dst_v, w_v, one_v, degv, cntv, zv, dbuf,
             deg_sh, cnt_sh):
        cid = lax.axis_index("c")
        sid = lax.axis_index("s")
        ones16 = jnp.full((LN,), 1.0, f32)
        zeros16 = jnp.zeros((LN,), f32)
        for k in range(BLK // LN):
            one_v[0, pl.ds(k * LN, LN)] = ones16

        def zv_body(k, carry):
            zv[pl.ds(k * LN, LN)] = zeros16
            return carry

        lax.fori_loop(0, RPT // LN, zv_body, 0)
        pltpu.sync_copy(zv, deg_sh.at[pl.ds(sid * RPT, RPT)])
        pltpu.sync_copy(zv, cnt_sh.at[pl.ds(sid * RPT, RPT)])
        pltpu.sync_copy(dst3.at[pl.ds(sid * 2, 2)], dst_v)
        pltpu.sync_copy(w3.at[pl.ds(sid * 2, 2)], w_v)
        plsc.subcore_barrier()
        for a in range(2):
            def deg_body(j, carry, a=a):
                pltpu.sync_copy(w_v.at[a, j], deg_sh.at[dst_v.at[a, j]], add=True)
                pltpu.sync_copy(one_v.at[0], cnt_sh.at[dst_v.at[a, j]], add=True)
                return carry

            lax.fori_loop(0, NBLK, deg_body, 0)
        plsc.subcore_barrier()
        pltpu.sync_copy(deg_sh, degv)
        pltpu.sync_copy(cnt_sh, cntv)

        def dinv_body(k, carry):
            sl = pl.ds(k * LN, LN)
            degv[sl] = _rsqrt16(degv[sl] + 1.0)
            cntv[sl] = _rsqrt16(cntv[sl] + 1.0)
            return carry

        lax.fori_loop(0, NR // LN, dinv_body, 0)

        @pl.when(cid == 0)
        def _():
            pltpu.sync_copy(cntv.at[pl.ds(sid * RPT, RPT)],
                            dinv5_o.at[pl.ds(sid * RPT, RPT)])

            def drow(m, carry):
                base = sid * RPT + m * LN
                d16 = degv[pl.ds(base, LN)]
                for rr in range(LN):
                    s = d16[rr]
                    for k in range(HID // LN):
                        dbuf[rr, pl.ds(k * LN, LN)] = s * ones16
                pltpu.sync_copy(dbuf, d_o.at[pl.ds(base, LN)])
                return carry

            lax.fori_loop(0, RPT // LN, drow, 0)

    return prep


def _make_mp():
    @functools.partial(
        pl.kernel,
        out_type=jax.ShapeDtypeStruct((NC, NR, HID), f32),
        mesh=_mesh(),
        scratch_types=[
            pltpu.VMEM((NBLK, BLK), i32),
            pltpu.VMEM((NBLK, BLK), i32),
            pltpu.VMEM((NBLK, BLK), f32),
            pltpu.VMEM((BLK, HID), f32),
            pltpu.VMEM((1, NBLK), i32),
            pltpu.VMEM((1, BLK), i32),
            pltpu.VMEM_SHARED((NR, HID), f32),
            pltpu.SemaphoreType.DMA,
        ],
        **_SC_PARAMS,
    )
    def mp(g, src2, dst2, w2, s_out,
           src_v, dst_v, w_v, rb0, idxw, idxb, acc, sem0):
        cid = lax.axis_index("c")
        sid = lax.axis_index("s")
        wid = cid * NS + sid
        iot = lax.iota(i32, LN)
        for k in range(NBLK // LN):
            idxw[0, pl.ds(k * LN, LN)] = wid * NBLK + k * LN + iot
        pltpu.async_copy(src2.at[idxw.at[0]], src_v, sem0).wait()
        pltpu.async_copy(dst2.at[idxw.at[0]], dst_v, sem0).wait()
        pltpu.async_copy(w2.at[idxw.at[0]], w_v, sem0).wait()

        @pl.when(cid == 1)
        def _():
            zeros16 = jnp.zeros((LN,), f32)

            def zrow(m, carry):
                for k in range(HID // LN):
                    rb0[m, pl.ds(k * LN, LN)] = zeros16
                return carry

            lax.fori_loop(0, BLK, zrow, 0)

        def initblk(jb, carry):
            base = sid * RPT + jb * BLK
            for k in range(BLK // LN):
                idxb[0, pl.ds(k * LN, LN)] = base + k * LN + iot

            @pl.when(cid == 0)
            def _():
                pltpu.async_copy(g.at[idxb.at[0]], rb0, sem0).wait()

            pltpu.sync_copy(rb0, acc.at[idxb.at[0]])
            return carry

        lax.fori_loop(0, RPT // BLK, initblk, 0)
        plsc.subcore_barrier()

        def scale(rb, j):
            def grp(m, carry):
                c16 = w_v[j, pl.ds(m * LN, LN)]
                for rr in range(LN):
                    sc = c16[rr]
                    r = m * LN + rr
                    for k in range(HID // LN):
                        sl = pl.ds(k * LN, LN)
                        rb[r, sl] = rb[r, sl] * sc
                return carry

            lax.fori_loop(0, BLK // LN, grp, 0)

        def step(j, carry):
            pltpu.async_copy(g.at[src_v.at[j]], rb0, sem0).wait()
            scale(rb0, j)
            pltpu.sync_copy(rb0, acc.at[dst_v.at[j]], add=True)
            return carry

        lax.fori_loop(0, NBLK, step, 0)
        plsc.subcore_barrier()
        sl2 = pl.ds(sid * RPT, RPT)
        pltpu.sync_copy(acc.at[sl2], s_out.at[cid, sl2])

    return mp


def _make_sc5():
    @functools.partial(
        pl.kernel,
        out_type=[
            jax.ShapeDtypeStruct((NO_PAD,), f32),
            jax.ShapeDtypeStruct((NR,), f32),
        ],
        mesh=_mesh(),
        scratch_types=[
            pltpu.VMEM((VPAD,), f32),
            pltpu.VMEM((NR,), f32),
            pltpu.VMEM((NR,), f32),
            pltpu.VMEM((NBLK, BLK), i32),
            pltpu.VMEM((NBLK, BLK), i32),
            pltpu.VMEM((1, BLK), f32),
            pltpu.VMEM((LN,), f32),
            pltpu.VMEM((LN,), f32),
            pltpu.VMEM((LN,), f32),
            pltpu.VMEM((1, BLK), f32),
            pltpu.VMEM_SHARED((NR,), f32),
        ],
        **_SC_PARAMS,
    )
    def sc5(h5, dinv5, src3, dst3, b5v, lwv, lbv, outp, p5b,
            g5v, d5v, accv, src_v, dst_v, val_v, b5_v, lw_v, lb_v, ov, acc5):
        cid = lax.axis_index("c")
        sid = lax.axis_index("s")
        pltpu.sync_copy(h5, g5v.at[pl.ds(0, NR)])
        pltpu.sync_copy(dinv5, d5v)
        pltpu.sync_copy(b5v, b5_v)
        pltpu.sync_copy(lwv, lw_v)
        pltpu.sync_copy(lbv, lb_v)
        zeros16 = jnp.zeros((LN,), f32)

        def gz(k, carry):
            sl = pl.ds(k * LN, LN)
            g5v[sl] = g5v[sl] * d5v[sl]
            return carry

        lax.fori_loop(0, NR // LN, gz, 0)

        def gz2(k, carry):
            g5v[pl.ds(NR + k * LN, LN)] = zeros16
            return carry

        lax.fori_loop(0, (VPAD - NR) // LN, gz2, 0)
        sl0 = pl.ds(sid * RPT, RPT)
        pltpu.sync_copy(g5v.at[sl0], acc5.at[sl0])
        plsc.subcore_barrier()
        for a in range(2):
            wrow = sid * 2 + a
            pltpu.sync_copy(src3.at[wrow], src_v)
            pltpu.sync_copy(dst3.at[wrow], dst_v)
            ebase = wrow * EW

            def eblk(j, carry, ebase=ebase):
                for k in range(BLK // LN):
                    sl = pl.ds(k * LN, LN)
                    sidx = src_v[j, sl]
                    g = plsc.load_gather(g5v, [sidx])
                    gidx = ebase + j * BLK + k * LN + lax.iota(i32, LN)
                    val_v[0, sl] = jnp.where(gidx < E, g, 0.0)
                pltpu.sync_copy(val_v.at[0], acc5.at[dst_v.at[j]], add=True)
                return carry

            lax.fori_loop(0, NBLK, eblk, 0)
        plsc.subcore_barrier()

        @pl.when(cid == 0)
        def _():
            pltpu.sync_copy(acc5.at[sl0], p5b.at[sl0])
            plsc.subcore_barrier()
            pltpu.sync_copy(p5b, accv)

            def vb(k, carry):
                sl = pl.ds(k * LN, LN)
                g5v[sl] = d5v[sl] * accv[sl] + b5_v[...]
                return carry

            lax.fori_loop(0, NR // LN, vb, 0)

            lw16 = lw_v[...]
            for t in range(BLK // LN):
                i16 = sid * BLK + t * LN + lax.iota(i32, LN)
                base6 = i16 * 6
                a16 = lb_v[...]
                for j in range(6):
                    g = plsc.load_gather(g5v, [base6 + j])
                    a16 = a16 + g * lw16[j]
                ov[0, pl.ds(t * LN, LN)] = a16
            pltpu.sync_copy(ov.at[0], outp.at[pl.ds(sid * BLK, BLK)])

    return sc5


def _tc1_body(x_ref, w_ref, d_ref, o_ref):
    o_ref[...] = d_ref[...] * jnp.dot(x_ref[...], w_ref[...],
                                      preferred_element_type=f32)


def _tc1(x_pad, w1p, D):
    RB = 2048
    return pl.pallas_call(
        _tc1_body,
        grid=(NR // RB,),
        in_specs=[pl.BlockSpec((RB, 8), lambda i: (i, 0)),
                  pl.BlockSpec((8, HID), lambda i: (0, 0)),
                  pl.BlockSpec((RB, HID), lambda i: (i, 0))],
        out_specs=pl.BlockSpec((RB, HID), lambda i: (i, 0)),
        out_shape=jax.ShapeDtypeStruct((NR, HID), f32),
    )(x_pad, w1p, D)


def _tcl_body(s_ref, d_ref, b_ref, a_ref, w_ref, o_ref):
    xsum = d_ref[...] * (s_ref[0] + s_ref[1]) + b_ref[...]
    a = a_ref[0]
    x = jnp.where(xsum >= 0, xsum, a * xsum)
    o_ref[...] = d_ref[...] * jnp.dot(x, w_ref[...],
                                      preferred_element_type=f32)


def _tc_layer(S, D, b, a_arr, W):
    RB = 2048
    return pl.pallas_call(
        _tcl_body,
        grid=(NR // RB,),
        in_specs=[pl.BlockSpec((NC, RB, HID), lambda i: (0, i, 0)),
                  pl.BlockSpec((RB, HID), lambda i: (i, 0)),
                  pl.BlockSpec((1, HID), lambda i: (0, 0)),
                  pl.BlockSpec(memory_space=pltpu.SMEM),
                  pl.BlockSpec((HID, HID), lambda i: (0, 0))],
        out_specs=pl.BlockSpec((RB, HID), lambda i: (i, 0)),
        out_shape=jax.ShapeDtypeStruct((NR, HID), f32),
    )(S, D, b.reshape(1, HID), a_arr, W)


def _tc4_body(s_ref, d_ref, b_ref, a_ref, w_ref, o_ref):
    xsum = d_ref[...] * (s_ref[0] + s_ref[1]) + b_ref[...]
    a = a_ref[0]
    x = jnp.where(xsum >= 0, xsum, a * xsum)
    for t in range(8):
        o_ref[t] = lax.dot_general(w_ref[...], x[t * BLK:(t + 1) * BLK, :],
                                   (((1,), (1,)), ((), ())),
                                   preferred_element_type=f32)


def _tc4(S, D, b, a_arr, w5t):
    RB = 1024
    return pl.pallas_call(
        _tc4_body,
        grid=(NR // RB,),
        in_specs=[pl.BlockSpec((NC, RB, HID), lambda i: (0, i, 0)),
                  pl.BlockSpec((RB, HID), lambda i: (i, 0)),
                  pl.BlockSpec((1, HID), lambda i: (0, 0)),
                  pl.BlockSpec(memory_space=pltpu.SMEM),
                  pl.BlockSpec((1, HID), lambda i: (0, 0))],
        out_specs=pl.BlockSpec((8, 1, HID), lambda i: (i, 0, 0)),
        out_shape=jax.ShapeDtypeStruct((NR // BLK, 1, HID), f32),
    )(S, D, b.reshape(1, HID), a_arr, w5t)


def kernel(x, edge_index, edge_weight, W1, b1, W2, b2, W3, b3, W4, b4,
           W5, b5, prelu_a, lin_W, lin_b):
    pad = EP - E
    padrow = (N + (jnp.arange(pad, dtype=i32) % (NR - N))).astype(i32)
    src3 = jnp.concatenate([edge_index[0], padrow]).reshape(NW, NBLK, BLK)
    dst3 = jnp.concatenate([edge_index[1], padrow]).reshape(NW, NBLK, BLK)
    w3 = jnp.concatenate(
        [edge_weight, jnp.zeros((pad,), f32)]).reshape(NW, NBLK, BLK)
    x_pad = jnp.zeros((NR, 8), f32).at[:N, :3].set(x)
    w1p = jnp.zeros((8, HID), f32).at[:3].set(W1)
    a_arr = prelu_a.reshape(1)
    b5v = jnp.full((LN,), b5[0], f32)
    lwv = jnp.zeros((LN,), f32).at[:6].set(lin_W[:, 0])
    lbv = jnp.full((LN,), lin_b[0], f32)

    src2 = src3.reshape(NW * NBLK, BLK)
    dst2 = dst3.reshape(NW * NBLK, BLK)
    w2 = w3.reshape(NW * NBLK, BLK)

    prep = _make_prep()
    mp = _make_mp()
    sc5 = _make_sc5()

    D, dinv5 = prep(dst3, w3)
    G = _tc1(x_pad, w1p, D)
    S = mp(G, src2, dst2, w2)
    G = _tc_layer(S, D, b1, a_arr, W2)
    S = mp(G, src2, dst2, w2)
    G = _tc_layer(S, D, b2, a_arr, W3)
    S = mp(G, src2, dst2, w2)
    G = _tc_layer(S, D, b3, a_arr, W4)
    S = mp(G, src2, dst2, w2)
    h5 = _tc4(S, D, b4, a_arr, W5.reshape(1, HID)).reshape(NR)
    outp, _ = sc5(h5, dinv5, src3, dst3, b5v, lwv, lbv)
    return outp[:NOUT]
